# baseline (speedup 1.0000x reference)
import jax
import jax.numpy as jnp
import numpy as np
from jax.experimental import pallas as pl


def _fused(x_ref, w0_ref, g_ref, k1_ref, kb_ref, k0_ref, k0b_ref,
           r14_ref, t_ref, w1r_ref, b1t_ref, k2_ref, b2t_ref, k3_ref,
           b3s_ref, o_ref):
    f32 = jnp.float32
    x = x_ref[...]
    w0 = w0_ref[...]
    g = g_ref[...]

    csw = jnp.sum(w0, axis=0, keepdims=True)
    rsw = jnp.sum(w0, axis=1, keepdims=True)
    csg = jnp.sum(g, axis=0, keepdims=True)
    rsh = jnp.dot(g, r14_ref[...], preferred_element_type=f32)

    rowvec = (jnp.dot(csw, k0b_ref[...], preferred_element_type=f32)
              + jnp.dot(csg, kb_ref[...], preferred_element_type=f32)
              + b1t_ref[...])
    pre1 = (jnp.dot(g, k1_ref[...], preferred_element_type=f32)
            + jnp.dot(w0, k0_ref[...], preferred_element_type=f32)
            + jnp.dot(rsh, t_ref[...], preferred_element_type=f32)
            + jnp.dot(rsw, w1r_ref[...], preferred_element_type=f32)
            + rowvec)
    a1 = jnp.maximum(pre1, 0.0)
    a2 = jnp.maximum(jnp.dot(a1, k2_ref[...], preferred_element_type=f32)
                     + b2t_ref[...], 0.0)
    upd = jnp.dot(a2, k3_ref[...], preferred_element_type=f32) + b3s_ref[...]
    new_w = w0 + upd

    logits = jnp.maximum(jnp.dot(x, new_w, preferred_element_type=f32), 0.0)
    m = jnp.max(logits, axis=1, keepdims=True)
    e = jnp.exp(logits - m)
    o_ref[...] = e / jnp.sum(e, axis=1, keepdims=True)


def kernel(X, w0, h0, W1, b1, W2, b2, W3, b3):
    in_u, out_u, H = h0.shape
    LH = W2.shape[0]
    batch = X.shape[0]
    inv_in = 1.0 / (in_u - 1)
    inv_out = 1.0 / (out_u - 1)

    w1_w = W1[0:1, :]
    w1_f = W1[1:2, :]
    w1_b = W1[2:3, :]
    A = W1[3:3 + H, :]
    B = W1[3 + H:3 + 2 * H, :]
    C = W1[3 + 2 * H:3 + 3 * H, :]

    eye_o = jnp.eye(out_u, dtype=jnp.float32)

    def kron_i(mat):
        return jnp.einsum('jk,hc->jhkc', eye_o, mat).reshape(
            out_u * mat.shape[0], out_u * mat.shape[1])

    K1 = kron_i(A - B * inv_in - C * inv_out)
    KB = kron_i(B * inv_in)
    K0 = kron_i(w1_w - w1_f * inv_in - w1_b * inv_out)
    K0b = kron_i(w1_f * inv_in)
    R14 = jnp.tile(jnp.eye(H, dtype=jnp.float32), (out_u, 1))
    T = jnp.tile(C * inv_out, (1, out_u))
    w1r = jnp.tile(w1_b * inv_out, (1, out_u))
    b1t = jnp.tile(b1[None, :], (1, out_u))
    K2 = kron_i(W2)
    b2t = jnp.tile(b2[None, :], (1, out_u))
    K3 = kron_i(W3[:, 0:1])
    b3s = jnp.full((1, 1), b3[0], dtype=jnp.float32)

    G = h0.reshape(in_u, out_u * H)

    return pl.pallas_call(
        _fused,
        out_shape=jax.ShapeDtypeStruct((batch, out_u), jnp.float32),
    )(X, w0, G, K1, KB, K0, K0b, R14, T, w1r, b1t, K2, b2t, K3, b3s)


# speedup vs baseline: 4.8816x; 4.8816x over previous
import jax
import jax.numpy as jnp
import numpy as np
from jax.experimental import pallas as pl
from jax.experimental.pallas import tpu as pltpu

_H = 14
_LH = 10


def _fused(x_ref, w0_ref, gt_ref, cext_ref, meff_ref, w1weff_ref,
           w1f_ref, bs_ref, b1_ref, w2_ref, b2_ref, w3_ref, o_ref):
    H, LH = _H, _LH
    in_u, out_u = w0_ref.shape
    f32 = jnp.float32

    w0 = w0_ref[...]
    planes = [gt_ref[h * in_u:(h + 1) * in_u, :] for h in range(H)]

    csw = jnp.sum(w0, axis=0, keepdims=True)
    csg = [jnp.sum(p, axis=0, keepdims=True) for p in planes]

    rs = [jnp.sum(p, axis=1, keepdims=True) for p in planes]
    rs.append(jnp.sum(w0, axis=1, keepdims=True))
    rsh = jnp.concatenate(rs, axis=1)
    rowcorr = jnp.dot(rsh, cext_ref[...], preferred_element_type=f32)

    a1 = []
    for c in range(LH):
        colrow = csw * w1f_ref[0, c] + b1_ref[0, c]
        for h in range(H):
            colrow = colrow + csg[h] * bs_ref[h, c]
        acc = w0 * w1weff_ref[0, c] + colrow + rowcorr[:, c:c + 1]
        for h in range(H):
            acc = acc + planes[h] * meff_ref[h, c]
        a1.append(jnp.maximum(acc, 0.0))

    a2 = []
    for d in range(LH):
        acc = a1[0] * w2_ref[0, d]
        for c in range(1, LH):
            acc = acc + a1[c] * w2_ref[c, d]
        a2.append(jnp.maximum(acc + b2_ref[0, d], 0.0))

    upd = a2[0] * w3_ref[0, 0]
    for d in range(1, LH):
        upd = upd + a2[d] * w3_ref[d, 0]
    new_w = w0 + upd + w3_ref[LH, 0]

    logits = jnp.maximum(
        jnp.dot(x_ref[...], new_w, preferred_element_type=f32), 0.0)
    m = jnp.max(logits, axis=1, keepdims=True)
    e = jnp.exp(logits - m)
    o_ref[...] = e / jnp.sum(e, axis=1, keepdims=True)


def kernel(X, w0, h0, W1, b1, W2, b2, W3, b3):
    in_u, out_u, H = h0.shape
    LH = W2.shape[0]
    inv_in = np.float32(1.0 / (in_u - 1))
    inv_out = np.float32(1.0 / (out_u - 1))

    w1_w, w1_f, w1_b = W1[0:1, :], W1[1:2, :], W1[2:3, :]
    A = W1[3:3 + H, :]
    B = W1[3 + H:3 + 2 * H, :]
    C = W1[3 + 2 * H:3 + 3 * H, :]

    meff = A - B * inv_in - C * inv_out
    w1weff = w1_w - w1_f * inv_in - w1_b * inv_out
    cext = jnp.concatenate([C * inv_out, w1_b * inv_out], axis=0)
    w3ext = jnp.concatenate([W3[:, 0:1], b3[0:1][:, None]], axis=0)

    gt = h0.transpose(2, 0, 1).reshape(H * in_u, out_u)

    smem = pl.BlockSpec(memory_space=pltpu.SMEM)
    vmem = pl.BlockSpec(memory_space=pltpu.VMEM)
    return pl.pallas_call(
        _fused,
        out_shape=jax.ShapeDtypeStruct((X.shape[0], out_u), jnp.float32),
        in_specs=[vmem, vmem, vmem, vmem,
                  smem, smem, smem, smem, smem, smem, smem, smem],
        out_specs=vmem,
    )(X, w0, gt, cext,
      meff, w1weff, w1_f * inv_in, B * inv_in,
      b1[None, :], W2, b2[None, :], w3ext)


# speedup vs baseline: 6.4538x; 1.3221x over previous
import jax
import jax.numpy as jnp
import numpy as np
from jax.experimental import pallas as pl
from jax.experimental.pallas import tpu as pltpu

_H = 14
_LH = 10


def _fused(x_ref, w0_ref, gt_ref, w1v_ref, w1s_ref, b1_ref, w2_ref,
           b2_ref, w3_ref, b3_ref, o_ref):
    H, LH = _H, _LH
    in_u, out_u = w0_ref.shape
    inv_in = np.float32(1.0 / (in_u - 1))
    inv_out = np.float32(1.0 / (out_u - 1))
    f32 = jnp.float32

    w0 = w0_ref[...]
    planes = [gt_ref[h * in_u:(h + 1) * in_u, :] for h in range(H)]

    csw = jnp.sum(w0, axis=0, keepdims=True)
    csg = [jnp.sum(p, axis=0, keepdims=True) for p in planes]

    rs = [jnp.sum(p, axis=1, keepdims=True) for p in planes]
    rs.append(jnp.sum(w0, axis=1, keepdims=True))
    rsh = jnp.concatenate(rs, axis=1)
    wrows = jnp.concatenate(
        [w1v_ref[3 + 2 * H:3 + 3 * H, :], w1v_ref[2:3, :]], axis=0)
    rowcorr = jnp.dot(rsh, wrows, preferred_element_type=f32) * inv_out

    a1 = []
    for c in range(LH):
        colrow = csw * (w1s_ref[1, c] * inv_in) + b1_ref[0, c]
        for h in range(H):
            colrow = colrow + csg[h] * (w1s_ref[3 + H + h, c] * inv_in)
        weff = (w1s_ref[0, c] - w1s_ref[1, c] * inv_in
                - w1s_ref[2, c] * inv_out)
        acc = w0 * weff + colrow + rowcorr[:, c:c + 1]
        for h in range(H):
            meff = (w1s_ref[3 + h, c] - w1s_ref[3 + H + h, c] * inv_in
                    - w1s_ref[3 + 2 * H + h, c] * inv_out)
            acc = acc + planes[h] * meff
        a1.append(jnp.maximum(acc, 0.0))

    a2 = []
    for d in range(LH):
        acc = a1[0] * w2_ref[0, d]
        for c in range(1, LH):
            acc = acc + a1[c] * w2_ref[c, d]
        a2.append(jnp.maximum(acc + b2_ref[0, d], 0.0))

    upd = a2[0] * w3_ref[0, 0]
    for d in range(1, LH):
        upd = upd + a2[d] * w3_ref[d, 0]
    new_w = w0 + upd + b3_ref[0, 0]

    logits = jnp.maximum(
        jnp.dot(x_ref[...], new_w, preferred_element_type=f32), 0.0)
    m = jnp.max(logits, axis=1, keepdims=True)
    e = jnp.exp(logits - m)
    o_ref[...] = e / jnp.sum(e, axis=1, keepdims=True)


def kernel(X, w0, h0, W1, b1, W2, b2, W3, b3):
    in_u, out_u, H = h0.shape
    gt = h0.transpose(2, 0, 1).reshape(H * in_u, out_u)

    smem = pl.BlockSpec(memory_space=pltpu.SMEM)
    vmem = pl.BlockSpec(memory_space=pltpu.VMEM)
    return pl.pallas_call(
        _fused,
        out_shape=jax.ShapeDtypeStruct((X.shape[0], out_u), jnp.float32),
        in_specs=[vmem, vmem, vmem, vmem,
                  smem, smem, smem, smem, smem, smem],
        out_specs=vmem,
    )(X, w0, gt, W1,
      W1, b1[None, :], W2, b2[None, :], W3, b3[None, :])
